# baseline (speedup 1.0000x reference)
import functools

import jax
import jax.numpy as jnp
from jax import lax
from jax.experimental import pallas as pl
from jax.experimental.pallas import tpu as pltpu
from jax.experimental.pallas import tpu_sc as plsc

_NODE_VOCABS = (120, 10, 12)
_EDGE_VOCABS = (6, 3)
_MASK_RATE = 0.15

_NC = 2
_NS = 16
_L = 16

_CH = 112
_HH = 32
_IB = 32


def _ceil_to(x, m):
    return (x + m - 1) // m * m


_NB = 2


def _msg_agg_kernel(NP, nchunk, rows_per_sub):
    mesh = plsc.VectorSubcoreMesh(core_axis_name="c", subcore_axis_name="s")

    nblk = nchunk // _IB

    def run_half(h, C, agg_out, src_r, ci_r, dst_r, zer, s,
                 srcb, cib, dstb, rows, cv, aggs, semg, sems):
        pltpu.sync_copy(zer, aggs.at[pl.ds(s * rows_per_sub, rows_per_sub)])
        pltpu.sync_copy(C, cv)
        row0 = s * nchunk
        plsc.subcore_barrier()

        def issue_gathers(k, b):
            pltpu.async_copy(h.at[srcb.at[k]], rows[b], semg[b])

        def block(bi, carry):
            @pl.when(bi > 0)
            def _():
                for b in range(_NB):
                    pltpu.make_async_copy(
                        rows[b], aggs.at[dstb.at[0]], sems[b]).wait()

            r0 = row0 + bi * _IB
            pltpu.sync_copy(src_r.at[pl.ds(r0, _IB)], srcb)
            pltpu.sync_copy(ci_r.at[pl.ds(r0, _IB)], cib)
            pltpu.sync_copy(dst_r.at[pl.ds(r0, _IB)], dstb)
            issue_gathers(0, 0)

            def group(gi, c1):
                for b in range(_NB):
                    k = gi * _NB + b
                    kf = k + 1
                    bf = (b + 1) % _NB

                    @pl.when(kf < _IB)
                    def _():
                        @pl.when(kf >= _NB)
                        def _():
                            pltpu.make_async_copy(
                                rows[bf], aggs.at[dstb.at[k]], sems[bf]).wait()
                        issue_gathers(kf, bf)

                    pltpu.make_async_copy(
                        h.at[srcb.at[k]], rows[b], semg[b]).wait()

                    for g in range(_CH // _L):
                        ci16 = cib[k, pl.ds(g * _L, _L)]
                        jvec = lax.iota(jnp.int32, _L) + (g * _L)
                        for d in range(_HH):
                            dspl = jnp.full((_L,), d, jnp.int32)
                            rv = plsc.load_gather(rows[b], [jvec, dspl])
                            cvv = plsc.load_gather(cv, [ci16, dspl])
                            m = jnp.maximum(rv + cvv, 0.0)
                            plsc.store_scatter(rows[b], [jvec, dspl], m)
                    pltpu.async_copy(rows[b], aggs.at[dstb.at[k]], sems[b],
                                     add=True)
                return c1

            lax.fori_loop(0, _IB // _NB, group, 0)
            return carry

        lax.fori_loop(0, nblk, block, 0)
        for b in range(_NB):
            pltpu.make_async_copy(rows[b], aggs.at[dstb.at[0]], sems[b]).wait()
        plsc.subcore_barrier()
        sl = pl.ds(s * rows_per_sub, rows_per_sub)
        pltpu.sync_copy(aggs.at[sl], agg_out.at[sl])

    @functools.partial(
        pl.kernel,
        out_type=(
            jax.ShapeDtypeStruct((NP, _HH), jnp.float32),
            jax.ShapeDtypeStruct((NP, _HH), jnp.float32),
        ),
        mesh=mesh,
        scratch_types=(
            pltpu.VMEM((_IB, _CH), jnp.int32),
            pltpu.VMEM((_IB, _CH), jnp.int32),
            pltpu.VMEM((_IB, _CH), jnp.int32),
            pltpu.VMEM((_CH, _HH), jnp.float32),
            pltpu.VMEM((_CH, _HH), jnp.float32),
            pltpu.VMEM((18, _HH), jnp.float32),
            pltpu.VMEM_SHARED((NP, _HH), jnp.float32),
            pltpu.SemaphoreType.DMA,
            pltpu.SemaphoreType.DMA,
            pltpu.SemaphoreType.DMA,
            pltpu.SemaphoreType.DMA,
        ),
        compiler_params=pltpu.CompilerParams(use_tc_tiling_on_sc=False,
                                             needs_layout_passes=False),
    )
    def kern(hA, hB, CA, CB, src_r, ci_r, dst_r, zer,
             aggA, aggB,
             srcb, cib, dstb,
             rows0, rows1, cv, aggs,
             semg0, semg1, sems0, sems1):
        c = lax.axis_index("c")
        s = lax.axis_index("s")
        rows = (rows0, rows1)
        semg = (semg0, semg1)
        sems = (sems0, sems1)

        @pl.when(c == 0)
        def _():
            run_half(hA, CA, aggA, src_r, ci_r, dst_r, zer, s,
                     srcb, cib, dstb, rows, cv, aggs, semg, sems)

        @pl.when(c == 1)
        def _():
            run_half(hB, CB, aggB, src_r, ci_r, dst_r, zer, s,
                     srcb, cib, dstb, rows, cv, aggs, semg, sems)

    return kern


def kernel(x, edge_index, edge_attr, params):
    N = x.shape[0]
    E = edge_attr.shape[0]
    H = params["node_emb"][0].shape[1]

    mkey = jax.random.key(42)
    node_mask = jax.random.uniform(jax.random.fold_in(mkey, 0), (N,)) < _MASK_RATE
    edge_mask = jax.random.uniform(jax.random.fold_in(mkey, 1), (E,)) < _MASK_RATE

    nfill = jnp.array([v - 1 for v in _NODE_VOCABS], dtype=x.dtype)
    efill = jnp.array([v - 1 for v in _EDGE_VOCABS], dtype=edge_attr.dtype)
    xm = jnp.where(node_mask[:, None], nfill[None, :], x)
    eam = jnp.where(edge_mask[:, None], efill[None, :], edge_attr)

    h = jnp.zeros((N, H), jnp.float32)
    for i, t in enumerate(params["node_emb"]):
        h = h + jnp.take(t, xm[:, i], axis=0)

    T0, T1 = params["edge_emb"]
    C = (T0[:, None, :] + T1[None, :, :]).reshape(
        _EDGE_VOCABS[0] * _EDGE_VOCABS[1], H)
    ci = eam[:, 0] * _EDGE_VOCABS[1] + eam[:, 1]

    per_sub = _ceil_to(-(-E // _NS), _CH * _IB)
    EP = per_sub * _NS
    nchunk = per_sub // _CH
    rows_per_sub = _ceil_to(-(-(N + 1) // _NS), 8)
    NP = rows_per_sub * _NS

    src = edge_index[0]
    dst = edge_index[1]
    pad_e = EP - E
    src_p = jnp.pad(src, (0, pad_e)).reshape(EP // _CH, _CH)
    ci_p = jnp.pad(ci, (0, pad_e)).reshape(EP // _CH, _CH)
    dst_p = jnp.pad(dst, (0, pad_e), constant_values=N).reshape(EP // _CH, _CH)
    zer = jnp.zeros((rows_per_sub, _HH), jnp.float32)
    CA, CB = C[:, :_HH], C[:, _HH:]

    sc_msg_agg = _msg_agg_kernel(NP, nchunk, rows_per_sub)

    for layer in params["layers"]:
        hp = jnp.pad(h, ((0, NP - N), (0, 0)))
        aggA, aggB = sc_msg_agg(hp[:, :_HH], hp[:, _HH:], CA, CB,
                                src_p, ci_p, dst_p, zer)
        agg = jnp.concatenate([aggA[:N], aggB[:N]], axis=1)
        z = h + agg
        z = jax.nn.relu(z @ layer["lin1"]["W"] + layer["lin1"]["b"])
        h = z @ layer["lin2"]["W"] + layer["lin2"]["b"]

    Wn = jnp.concatenate([hd["W"] for hd in params["node_heads"]], axis=1)
    bn = jnp.concatenate([hd["b"] for hd in params["node_heads"]])
    node_pred = h @ Wn + bn

    We = jnp.concatenate([hd["W"] for hd in params["edge_heads"]], axis=1)
    be = jnp.concatenate([hd["b"] for hd in params["edge_heads"]])
    Q = h @ We
    edge_pred = jnp.take(Q, src, axis=0) + jnp.take(Q, dst, axis=0) + be

    return node_pred, edge_pred, node_mask, edge_mask


# speedup vs baseline: 1.2697x; 1.2697x over previous
import functools

import jax
import jax.numpy as jnp
from jax import lax
from jax.experimental import pallas as pl
from jax.experimental.pallas import tpu as pltpu
from jax.experimental.pallas import tpu_sc as plsc

_NODE_VOCABS = (120, 10, 12)
_EDGE_VOCABS = (6, 3)
_MASK_RATE = 0.15

_NC = 2
_NS = 16
_L = 16

_CH = 112
_HH = 32
_IB = 32
_GS = 4
_CQ = _CH // _GS


def _ceil_to(x, m):
    return (x + m - 1) // m * m


_NB = 2


def _msg_agg_kernel(NP, nchunk, rows_per_sub):
    mesh = plsc.VectorSubcoreMesh(core_axis_name="c", subcore_axis_name="s")

    nblk = nchunk // _IB

    def run_half(h, C, agg_out, src_r, ci_r, dst_r, zer, s,
                 srcb, cib, dstb, rows, cv, aggs, semg, sems):
        pltpu.sync_copy(zer, aggs.at[pl.ds(s * rows_per_sub, rows_per_sub)])
        pltpu.sync_copy(C, cv)
        row0 = s * nchunk
        plsc.subcore_barrier()

        def issue_gathers(k, b):
            for q in range(_GS):
                pltpu.async_copy(h.at[srcb.at[k * _GS + q]],
                                 rows[b].at[pl.ds(q * _CQ, _CQ)], semg[b])

        def drain_slot(b):
            for q in range(_GS):
                pltpu.make_async_copy(rows[b].at[pl.ds(q * _CQ, _CQ)],
                                      aggs.at[dstb.at[q]], sems[b]).wait()

        def block(bi, carry):
            @pl.when(bi > 0)
            def _():
                for b in range(_NB):
                    drain_slot(b)

            r0 = row0 + bi * _IB
            pltpu.sync_copy(src_r.at[pl.ds(r0 * _GS, _IB * _GS)], srcb)
            pltpu.sync_copy(ci_r.at[pl.ds(r0, _IB)], cib)
            pltpu.sync_copy(dst_r.at[pl.ds(r0 * _GS, _IB * _GS)], dstb)
            issue_gathers(0, 0)

            def group(gi, c1):
                for b in range(_NB):
                    k = gi * _NB + b
                    kf = k + 1
                    bf = (b + 1) % _NB

                    @pl.when(kf < _IB)
                    def _():
                        @pl.when(kf >= _NB)
                        def _():
                            drain_slot(bf)
                        issue_gathers(kf, bf)

                    for q in range(_GS):
                        pltpu.make_async_copy(
                            h.at[srcb.at[k * _GS + q]],
                            rows[b].at[pl.ds(q * _CQ, _CQ)], semg[b]).wait()

                    for g in range(_CH // _L):
                        ci16 = cib[k, pl.ds(g * _L, _L)]
                        jvec = lax.iota(jnp.int32, _L) + (g * _L)
                        for d in range(_HH):
                            dspl = jnp.full((_L,), d, jnp.int32)
                            rv = plsc.load_gather(rows[b], [jvec, dspl])
                            cvv = plsc.load_gather(cv, [ci16, dspl])
                            m = jnp.maximum(rv + cvv, 0.0)
                            plsc.store_scatter(rows[b], [jvec, dspl], m)
                    for q in range(_GS):
                        pltpu.async_copy(rows[b].at[pl.ds(q * _CQ, _CQ)],
                                         aggs.at[dstb.at[k * _GS + q]],
                                         sems[b], add=True)
                return c1

            lax.fori_loop(0, _IB // _NB, group, 0)
            return carry

        lax.fori_loop(0, nblk, block, 0)
        for b in range(_NB):
            drain_slot(b)
        plsc.subcore_barrier()
        sl = pl.ds(s * rows_per_sub, rows_per_sub)
        pltpu.sync_copy(aggs.at[sl], agg_out.at[sl])

    @functools.partial(
        pl.kernel,
        out_type=(
            jax.ShapeDtypeStruct((NP, _HH), jnp.float32),
            jax.ShapeDtypeStruct((NP, _HH), jnp.float32),
        ),
        mesh=mesh,
        scratch_types=(
            pltpu.VMEM((_IB * _GS, _CQ), jnp.int32),
            pltpu.VMEM((_IB, _CH), jnp.int32),
            pltpu.VMEM((_IB * _GS, _CQ), jnp.int32),
            pltpu.VMEM((_CH, _HH), jnp.float32),
            pltpu.VMEM((_CH, _HH), jnp.float32),
            pltpu.VMEM((18, _HH), jnp.float32),
            pltpu.VMEM_SHARED((NP, _HH), jnp.float32),
            pltpu.SemaphoreType.DMA,
            pltpu.SemaphoreType.DMA,
            pltpu.SemaphoreType.DMA,
            pltpu.SemaphoreType.DMA,
        ),
        compiler_params=pltpu.CompilerParams(use_tc_tiling_on_sc=False,
                                             needs_layout_passes=False),
    )
    def kern(hA, hB, CA, CB, src_r, ci_r, dst_r, zer,
             aggA, aggB,
             srcb, cib, dstb,
             rows0, rows1, cv, aggs,
             semg0, semg1, sems0, sems1):
        c = lax.axis_index("c")
        s = lax.axis_index("s")
        rows = (rows0, rows1)
        semg = (semg0, semg1)
        sems = (sems0, sems1)

        @pl.when(c == 0)
        def _():
            run_half(hA, CA, aggA, src_r, ci_r, dst_r, zer, s,
                     srcb, cib, dstb, rows, cv, aggs, semg, sems)

        @pl.when(c == 1)
        def _():
            run_half(hB, CB, aggB, src_r, ci_r, dst_r, zer, s,
                     srcb, cib, dstb, rows, cv, aggs, semg, sems)

    return kern


def kernel(x, edge_index, edge_attr, params):
    N = x.shape[0]
    E = edge_attr.shape[0]
    H = params["node_emb"][0].shape[1]

    mkey = jax.random.key(42)
    node_mask = jax.random.uniform(jax.random.fold_in(mkey, 0), (N,)) < _MASK_RATE
    edge_mask = jax.random.uniform(jax.random.fold_in(mkey, 1), (E,)) < _MASK_RATE

    nfill = jnp.array([v - 1 for v in _NODE_VOCABS], dtype=x.dtype)
    efill = jnp.array([v - 1 for v in _EDGE_VOCABS], dtype=edge_attr.dtype)
    xm = jnp.where(node_mask[:, None], nfill[None, :], x)
    eam = jnp.where(edge_mask[:, None], efill[None, :], edge_attr)

    h = jnp.zeros((N, H), jnp.float32)
    for i, t in enumerate(params["node_emb"]):
        h = h + jnp.take(t, xm[:, i], axis=0)

    T0, T1 = params["edge_emb"]
    C = (T0[:, None, :] + T1[None, :, :]).reshape(
        _EDGE_VOCABS[0] * _EDGE_VOCABS[1], H)
    ci = eam[:, 0] * _EDGE_VOCABS[1] + eam[:, 1]

    per_sub = _ceil_to(-(-E // _NS), _CH * _IB)
    EP = per_sub * _NS
    nchunk = per_sub // _CH
    rows_per_sub = _ceil_to(-(-(N + 1) // _NS), 8)
    NP = rows_per_sub * _NS

    src = edge_index[0]
    dst = edge_index[1]
    pad_e = EP - E
    src_p = jnp.pad(src, (0, pad_e)).reshape(EP // _CQ, _CQ)
    ci_p = jnp.pad(ci, (0, pad_e)).reshape(EP // _CH, _CH)
    dst_p = jnp.pad(dst, (0, pad_e), constant_values=N).reshape(EP // _CQ, _CQ)
    zer = jnp.zeros((rows_per_sub, _HH), jnp.float32)
    CA, CB = C[:, :_HH], C[:, _HH:]

    sc_msg_agg = _msg_agg_kernel(NP, nchunk, rows_per_sub)

    for layer in params["layers"]:
        hp = jnp.pad(h, ((0, NP - N), (0, 0)))
        aggA, aggB = sc_msg_agg(hp[:, :_HH], hp[:, _HH:], CA, CB,
                                src_p, ci_p, dst_p, zer)
        agg = jnp.concatenate([aggA[:N], aggB[:N]], axis=1)
        z = h + agg
        z = jax.nn.relu(z @ layer["lin1"]["W"] + layer["lin1"]["b"])
        h = z @ layer["lin2"]["W"] + layer["lin2"]["b"]

    Wn = jnp.concatenate([hd["W"] for hd in params["node_heads"]], axis=1)
    bn = jnp.concatenate([hd["b"] for hd in params["node_heads"]])
    node_pred = h @ Wn + bn

    We = jnp.concatenate([hd["W"] for hd in params["edge_heads"]], axis=1)
    be = jnp.concatenate([hd["b"] for hd in params["edge_heads"]])
    Q = h @ We
    edge_pred = jnp.take(Q, src, axis=0) + jnp.take(Q, dst, axis=0) + be

    return node_pred, edge_pred, node_mask, edge_mask


# speedup vs baseline: 1.2698x; 1.0001x over previous
import functools

import jax
import jax.numpy as jnp
from jax import lax
from jax.experimental import pallas as pl
from jax.experimental.pallas import tpu as pltpu
from jax.experimental.pallas import tpu_sc as plsc

_NODE_VOCABS = (120, 10, 12)
_EDGE_VOCABS = (6, 3)
_MASK_RATE = 0.15

_NC = 2
_NS = 16
_L = 16

_CH = 112
_HH = 32
_IB = 32
_GS = 4
_CQ = _CH // _GS


def _ceil_to(x, m):
    return (x + m - 1) // m * m


_NB = 2


def _msg_agg_kernel(NP, nchunk, rows_per_sub):
    mesh = plsc.VectorSubcoreMesh(core_axis_name="c", subcore_axis_name="s")

    nblk = nchunk // _IB

    def run_half(h, C, agg_out, src_r, ci_r, dst_r, zer, s,
                 srcb, cib, dstb, rows, cv, aggs, semg, sems):
        pltpu.sync_copy(zer, aggs.at[pl.ds(s * rows_per_sub, rows_per_sub)])
        pltpu.sync_copy(C, cv)
        row0 = s * nchunk
        plsc.subcore_barrier()

        def issue_gathers(k, b):
            for q in range(_GS):
                pltpu.async_copy(h.at[srcb.at[k * _GS + q]],
                                 rows[b].at[pl.ds(q * _CQ, _CQ)], semg[b])

        def drain_slot(b):
            for q in range(_GS):
                pltpu.make_async_copy(rows[b].at[pl.ds(q * _CQ, _CQ)],
                                      aggs.at[dstb.at[q]], sems[b]).wait()

        def block(bi, carry):
            @pl.when(bi > 0)
            def _():
                for b in range(_NB):
                    drain_slot(b)

            r0 = row0 + bi * _IB
            pltpu.sync_copy(src_r.at[pl.ds(r0 * _GS, _IB * _GS)], srcb)
            pltpu.sync_copy(ci_r.at[pl.ds(r0, _IB)], cib)
            pltpu.sync_copy(dst_r.at[pl.ds(r0 * _GS, _IB * _GS)], dstb)
            issue_gathers(0, 0)

            def group(gi, c1):
                for b in range(_NB):
                    k = gi * _NB + b
                    kf = k + 1
                    bf = (b + 1) % _NB

                    @pl.when(kf < _IB)
                    def _():
                        @pl.when(kf >= _NB)
                        def _():
                            drain_slot(bf)
                        issue_gathers(kf, bf)

                    for q in range(_GS):
                        pltpu.make_async_copy(
                            h.at[srcb.at[k * _GS + q]],
                            rows[b].at[pl.ds(q * _CQ, _CQ)], semg[b]).wait()

                    for g in range(_CH // _L):
                        ci16 = cib[k, pl.ds(g * _L, _L)]
                        jvec = lax.iota(jnp.int32, _L) + (g * _L)
                        for d in range(_HH):
                            dspl = jnp.full((_L,), d, jnp.int32)
                            rv = plsc.load_gather(rows[b], [jvec, dspl])
                            cvv = plsc.load_gather(cv, [ci16, dspl])
                            m = jnp.maximum(rv + cvv, 0.0)
                            plsc.store_scatter(rows[b], [jvec, dspl], m)
                    for q in range(_GS):
                        pltpu.async_copy(rows[b].at[pl.ds(q * _CQ, _CQ)],
                                         aggs.at[dstb.at[k * _GS + q]],
                                         sems[b], add=True)
                return c1

            lax.fori_loop(0, _IB // _NB, group, 0)
            return carry

        lax.fori_loop(0, nblk, block, 0)
        for b in range(_NB):
            drain_slot(b)
        plsc.subcore_barrier()
        sl = pl.ds(s * rows_per_sub, rows_per_sub)
        pltpu.sync_copy(aggs.at[sl], agg_out.at[sl])

    @functools.partial(
        pl.kernel,
        out_type=(
            jax.ShapeDtypeStruct((NP, _HH), jnp.float32),
            jax.ShapeDtypeStruct((NP, _HH), jnp.float32),
        ),
        mesh=mesh,
        scratch_types=(
            pltpu.VMEM((_IB * _GS, _CQ), jnp.int32),
            pltpu.VMEM((_IB, _CH), jnp.int32),
            pltpu.VMEM((_IB * _GS, _CQ), jnp.int32),
            pltpu.VMEM((_CH, _HH), jnp.float32),
            pltpu.VMEM((_CH, _HH), jnp.float32),
            pltpu.VMEM((18, _HH), jnp.float32),
            pltpu.VMEM_SHARED((NP, _HH), jnp.float32),
            pltpu.SemaphoreType.DMA,
            pltpu.SemaphoreType.DMA,
            pltpu.SemaphoreType.DMA,
            pltpu.SemaphoreType.DMA,
        ),
        compiler_params=pltpu.CompilerParams(use_tc_tiling_on_sc=False,
                                             needs_layout_passes=False),
    )
    def kern(hA, hB, CA, CB, src_r, ci_r, dst_r, zer,
             aggA, aggB,
             srcb, cib, dstb,
             rows0, rows1, cv, aggs,
             semg0, semg1, sems0, sems1):
        c = lax.axis_index("c")
        s = lax.axis_index("s")
        rows = (rows0, rows1)
        semg = (semg0, semg1)
        sems = (sems0, sems1)

        @pl.when(c == 0)
        def _():
            run_half(hA, CA, aggA, src_r, ci_r, dst_r, zer, s,
                     srcb, cib, dstb, rows, cv, aggs, semg, sems)

        @pl.when(c == 1)
        def _():
            run_half(hB, CB, aggB, src_r, ci_r, dst_r, zer, s,
                     srcb, cib, dstb, rows, cv, aggs, semg, sems)

    return kern


_RB = 512


def _mlp_mid_call(NP):
    def body(hA_r, hB_r, aA_r, aB_r, W1_r, b1_r, W2_r, b2_r, oA_r, oB_r):
        zA = hA_r[...] + aA_r[...]
        zB = hB_r[...] + aB_r[...]
        W1m = W1_r[...]
        u = jnp.maximum(
            jnp.dot(zA, W1m[:_HH], preferred_element_type=jnp.float32)
            + jnp.dot(zB, W1m[_HH:], preferred_element_type=jnp.float32)
            + b1_r[...], 0.0)
        hn = jnp.dot(u, W2_r[...], preferred_element_type=jnp.float32) + b2_r[...]
        oA_r[...] = hn[:, :_HH]
        oB_r[...] = hn[:, _HH:]

    return pl.pallas_call(
        body,
        grid=(NP // _RB,),
        in_specs=[
            pl.BlockSpec((_RB, _HH), lambda i: (i, 0)),
            pl.BlockSpec((_RB, _HH), lambda i: (i, 0)),
            pl.BlockSpec((_RB, _HH), lambda i: (i, 0)),
            pl.BlockSpec((_RB, _HH), lambda i: (i, 0)),
            pl.BlockSpec((2 * _HH, 4 * _HH), lambda i: (0, 0)),
            pl.BlockSpec((1, 4 * _HH), lambda i: (0, 0)),
            pl.BlockSpec((4 * _HH, 2 * _HH), lambda i: (0, 0)),
            pl.BlockSpec((1, 2 * _HH), lambda i: (0, 0)),
        ],
        out_specs=[pl.BlockSpec((_RB, _HH), lambda i: (i, 0))] * 2,
        out_shape=[jax.ShapeDtypeStruct((NP, _HH), jnp.float32)] * 2,
    )


def _mlp_last_call(NP, NV):
    def body(hA_r, hB_r, aA_r, aB_r, W1_r, b1_r, W2_r, b2_r,
             Wn_r, bn_r, We_r, be_r, np_r, q_r, qb_r):
        zA = hA_r[...] + aA_r[...]
        zB = hB_r[...] + aB_r[...]
        W1m = W1_r[...]
        u = jnp.maximum(
            jnp.dot(zA, W1m[:_HH], preferred_element_type=jnp.float32)
            + jnp.dot(zB, W1m[_HH:], preferred_element_type=jnp.float32)
            + b1_r[...], 0.0)
        hn = jnp.dot(u, W2_r[...], preferred_element_type=jnp.float32) + b2_r[...]
        np_r[...] = jnp.dot(hn, Wn_r[...],
                            preferred_element_type=jnp.float32) + bn_r[...]
        q = jnp.dot(hn, We_r[...], preferred_element_type=jnp.float32)
        q_r[...] = q
        qb_r[...] = q + be_r[...]

    return pl.pallas_call(
        body,
        grid=(NP // _RB,),
        in_specs=[
            pl.BlockSpec((_RB, _HH), lambda i: (i, 0)),
            pl.BlockSpec((_RB, _HH), lambda i: (i, 0)),
            pl.BlockSpec((_RB, _HH), lambda i: (i, 0)),
            pl.BlockSpec((_RB, _HH), lambda i: (i, 0)),
            pl.BlockSpec((2 * _HH, 4 * _HH), lambda i: (0, 0)),
            pl.BlockSpec((1, 4 * _HH), lambda i: (0, 0)),
            pl.BlockSpec((4 * _HH, 2 * _HH), lambda i: (0, 0)),
            pl.BlockSpec((1, 2 * _HH), lambda i: (0, 0)),
            pl.BlockSpec((2 * _HH, NV), lambda i: (0, 0)),
            pl.BlockSpec((1, NV), lambda i: (0, 0)),
            pl.BlockSpec((2 * _HH, 16), lambda i: (0, 0)),
            pl.BlockSpec((1, 16), lambda i: (0, 0)),
        ],
        out_specs=[
            pl.BlockSpec((_RB, NV), lambda i: (i, 0)),
            pl.BlockSpec((_RB, 16), lambda i: (i, 0)),
            pl.BlockSpec((_RB, 16), lambda i: (i, 0)),
        ],
        out_shape=[
            jax.ShapeDtypeStruct((NP, NV), jnp.float32),
            jax.ShapeDtypeStruct((NP, 16), jnp.float32),
            jax.ShapeDtypeStruct((NP, 16), jnp.float32),
        ],
    )


def _node_prep_call(R):
    off1 = _NODE_VOCABS[0]
    off2 = _NODE_VOCABS[0] + _NODE_VOCABS[1]

    def body(x0r, x1r, x2r, mr, o0, o1, o2):
        m = mr[...] == 1
        o0[...] = jnp.where(m, _NODE_VOCABS[0] - 1, x0r[...])
        o1[...] = jnp.where(m, _NODE_VOCABS[1] - 1, x1r[...]) + off1
        o2[...] = jnp.where(m, _NODE_VOCABS[2] - 1, x2r[...]) + off2

    return pl.pallas_call(
        body,
        grid=(1,),
        in_specs=[pl.BlockSpec((R, 128), lambda i: (0, 0))] * 4,
        out_specs=[pl.BlockSpec((R, 128), lambda i: (0, 0))] * 3,
        out_shape=[jax.ShapeDtypeStruct((R, 128), jnp.int32)] * 3,
    )


def _edge_prep_call(R):
    def body(a0r, a1r, mr, o):
        m = mr[...] == 1
        ci = a0r[...] * _EDGE_VOCABS[1] + a1r[...]
        o[...] = jnp.where(m, _EDGE_VOCABS[0] * _EDGE_VOCABS[1] - 1, ci)

    return pl.pallas_call(
        body,
        grid=(R // 784,),
        in_specs=[pl.BlockSpec((784, 128), lambda i: (i, 0))] * 3,
        out_specs=pl.BlockSpec((784, 128), lambda i: (i, 0)),
        out_shape=jax.ShapeDtypeStruct((R, 128), jnp.int32),
    )


def _node_embed_kernel(NP):
    mesh = plsc.VectorSubcoreMesh(core_axis_name="c", subcore_axis_name="s")
    NCH = NP // _CH
    PW = NCH // _NS
    NVOC = sum(_NODE_VOCABS)

    def run_half(T, out, x0_r, x1_r, x2_r, s, x0b, x1b, x2b, tv, hbuf):
        pltpu.sync_copy(T, tv)
        r0 = s * PW
        pltpu.sync_copy(x0_r.at[pl.ds(r0, PW)], x0b)
        pltpu.sync_copy(x1_r.at[pl.ds(r0, PW)], x1b)
        pltpu.sync_copy(x2_r.at[pl.ds(r0, PW)], x2b)

        def chunk(k, carry):
            for g in range(_CH // _L):
                i0 = x0b[k, pl.ds(g * _L, _L)]
                i1 = x1b[k, pl.ds(g * _L, _L)]
                i2 = x2b[k, pl.ds(g * _L, _L)]
                jvec = lax.iota(jnp.int32, _L) + (g * _L)
                for d in range(_HH):
                    dspl = jnp.full((_L,), d, jnp.int32)
                    v = (plsc.load_gather(tv, [i0, dspl])
                         + plsc.load_gather(tv, [i1, dspl])
                         + plsc.load_gather(tv, [i2, dspl]))
                    plsc.store_scatter(hbuf, [jvec, dspl], v)
            pltpu.sync_copy(hbuf, out.at[pl.ds((r0 + k) * _CH, _CH)])
            return carry

        lax.fori_loop(0, PW, chunk, 0)

    @functools.partial(
        pl.kernel,
        out_type=(
            jax.ShapeDtypeStruct((NP, _HH), jnp.float32),
            jax.ShapeDtypeStruct((NP, _HH), jnp.float32),
        ),
        mesh=mesh,
        scratch_types=(
            pltpu.VMEM((PW, _CH), jnp.int32),
            pltpu.VMEM((PW, _CH), jnp.int32),
            pltpu.VMEM((PW, _CH), jnp.int32),
            pltpu.VMEM((NVOC, _HH), jnp.float32),
            pltpu.VMEM((_CH, _HH), jnp.float32),
        ),
        compiler_params=pltpu.CompilerParams(use_tc_tiling_on_sc=False,
                                             needs_layout_passes=False),
    )
    def kern(x0_r, x1_r, x2_r, TA, TB, hA, hB, x0b, x1b, x2b, tv, hbuf):
        c = lax.axis_index("c")
        s = lax.axis_index("s")

        @pl.when(c == 0)
        def _():
            run_half(TA, hA, x0_r, x1_r, x2_r, s, x0b, x1b, x2b, tv, hbuf)

        @pl.when(c == 1)
        def _():
            run_half(TB, hB, x0_r, x1_r, x2_r, s, x0b, x1b, x2b, tv, hbuf)

    return kern


def _edge_pred_kernel(NP, EP):
    mesh = plsc.VectorSubcoreMesh(core_axis_name="c", subcore_axis_name="s")
    NCH = EP // _CH
    PW = NCH // (_NC * _NS)
    EB = 32
    nblk = PW // EB

    @functools.partial(
        pl.kernel,
        out_type=jax.ShapeDtypeStruct((EP, 16), jnp.float32),
        mesh=mesh,
        scratch_types=(
            pltpu.VMEM((EB * _GS, _CQ), jnp.int32),
            pltpu.VMEM((EB * _GS, _CQ), jnp.int32),
            pltpu.VMEM((_CH, 16), jnp.float32),
            pltpu.VMEM((_CH, 16), jnp.float32),
            pltpu.VMEM((_CH, 16), jnp.float32),
            pltpu.VMEM((_CH, 16), jnp.float32),
            pltpu.SemaphoreType.DMA,
            pltpu.SemaphoreType.DMA,
            pltpu.SemaphoreType.DMA,
            pltpu.SemaphoreType.DMA,
        ),
        compiler_params=pltpu.CompilerParams(use_tc_tiling_on_sc=False,
                                             needs_layout_passes=False),
    )
    def kern(Q, Qb, src_r, dst_r, ep,
             srcb, dstb, qs0, qs1, qd0, qd1, semg0, semg1, semo0, semo1):
        c = lax.axis_index("c")
        s = lax.axis_index("s")
        w = s * _NC + c
        qs = (qs0, qs1)
        qd = (qd0, qd1)
        semg = (semg0, semg1)
        semo = (semo0, semo1)
        row0 = w * PW

        def issue_gathers(k, b):
            for q in range(_GS):
                pltpu.async_copy(Q.at[srcb.at[k * _GS + q]],
                                 qs[b].at[pl.ds(q * _CQ, _CQ)], semg[b])
                pltpu.async_copy(Qb.at[dstb.at[k * _GS + q]],
                                 qd[b].at[pl.ds(q * _CQ, _CQ)], semg[b])

        def block(bi, carry):
            @pl.when(bi > 0)
            def _():
                for b in range(_NB):
                    pltpu.make_async_copy(
                        qs[b], ep.at[pl.ds(0, _CH)], semo[b]).wait()

            r0 = row0 + bi * EB
            pltpu.sync_copy(src_r.at[pl.ds(r0 * _GS, EB * _GS)], srcb)
            pltpu.sync_copy(dst_r.at[pl.ds(r0 * _GS, EB * _GS)], dstb)
            issue_gathers(0, 0)

            def group(gi, c1):
                for b in range(_NB):
                    k = gi * _NB + b
                    kf = k + 1
                    bf = (b + 1) % _NB

                    @pl.when(kf < EB)
                    def _():
                        @pl.when(kf >= _NB)
                        def _():
                            pltpu.make_async_copy(
                                qs[bf], ep.at[pl.ds(0, _CH)], semo[bf]).wait()
                        issue_gathers(kf, bf)

                    for q in range(_GS):
                        pltpu.make_async_copy(
                            Q.at[srcb.at[k * _GS + q]],
                            qs[b].at[pl.ds(q * _CQ, _CQ)], semg[b]).wait()
                        pltpu.make_async_copy(
                            Qb.at[dstb.at[k * _GS + q]],
                            qd[b].at[pl.ds(q * _CQ, _CQ)], semg[b]).wait()

                    def jbody(j, c2):
                        qs[b][j, pl.ds(0, _L)] = (qs[b][j, pl.ds(0, _L)]
                                                  + qd[b][j, pl.ds(0, _L)])
                        return c2

                    lax.fori_loop(0, _CH, jbody, 0, unroll=8)
                    cr = row0 + bi * EB + k
                    pltpu.async_copy(qs[b], ep.at[pl.ds(cr * _CH, _CH)],
                                     semo[b])
                return c1

            lax.fori_loop(0, EB // _NB, group, 0)
            return carry

        lax.fori_loop(0, nblk, block, 0)
        for b in range(_NB):
            pltpu.make_async_copy(qs[b], ep.at[pl.ds(0, _CH)], semo[b]).wait()

    return kern


def kernel(x, edge_index, edge_attr, params):
    N = x.shape[0]
    E = edge_attr.shape[0]
    H = params["node_emb"][0].shape[1]

    mkey = jax.random.key(42)
    node_mask = jax.random.uniform(jax.random.fold_in(mkey, 0), (N,)) < _MASK_RATE
    edge_mask = jax.random.uniform(jax.random.fold_in(mkey, 1), (E,)) < _MASK_RATE

    per_sub = _ceil_to(-(-E // _NS), _CH * _IB)
    EP = per_sub * _NS
    nchunk = per_sub // _CH
    rows_per_sub = 3136
    NP = rows_per_sub * _NS

    NR = NP // 128
    nm32 = jnp.pad(node_mask.astype(jnp.int32), (0, NP - N)).reshape(NR, 128)
    x0 = jnp.pad(x[:, 0], (0, NP - N)).reshape(NR, 128)
    x1 = jnp.pad(x[:, 1], (0, NP - N)).reshape(NR, 128)
    x2 = jnp.pad(x[:, 2], (0, NP - N)).reshape(NR, 128)
    x0m, x1m, x2m = _node_prep_call(NR)(x0, x1, x2, nm32)

    ER = EP // 128
    pad_e = EP - E
    em32 = jnp.pad(edge_mask.astype(jnp.int32), (0, pad_e)).reshape(ER, 128)
    ea0 = jnp.pad(edge_attr[:, 0], (0, pad_e)).reshape(ER, 128)
    ea1 = jnp.pad(edge_attr[:, 1], (0, pad_e)).reshape(ER, 128)
    ci_p = _edge_prep_call(ER)(ea0, ea1, em32).reshape(EP // _CH, _CH)

    Tstk = jnp.concatenate(params["node_emb"], axis=0)
    hA, hB = _node_embed_kernel(NP)(
        x0m.reshape(NP // _CH, _CH), x1m.reshape(NP // _CH, _CH),
        x2m.reshape(NP // _CH, _CH), Tstk[:, :_HH], Tstk[:, _HH:])

    T0, T1 = params["edge_emb"]
    C = (T0[:, None, :] + T1[None, :, :]).reshape(
        _EDGE_VOCABS[0] * _EDGE_VOCABS[1], H)
    CA, CB = C[:, :_HH], C[:, _HH:]

    src = edge_index[0]
    dst = edge_index[1]
    src_p = jnp.pad(src, (0, pad_e)).reshape(EP // _CQ, _CQ)
    dst_p = jnp.pad(dst, (0, pad_e), constant_values=N).reshape(EP // _CQ, _CQ)
    zer = jnp.zeros((rows_per_sub, _HH), jnp.float32)

    sc_msg_agg = _msg_agg_kernel(NP, nchunk, rows_per_sub)
    mlp_mid = _mlp_mid_call(NP)

    for layer in params["layers"][:-1]:
        aggA, aggB = sc_msg_agg(hA, hB, CA, CB, src_p, ci_p, dst_p, zer)
        hA, hB = mlp_mid(hA, hB, aggA, aggB,
                         layer["lin1"]["W"], layer["lin1"]["b"][None, :],
                         layer["lin2"]["W"], layer["lin2"]["b"][None, :])

    Wn = jnp.concatenate([hd["W"] for hd in params["node_heads"]], axis=1)
    bn = jnp.concatenate([hd["b"] for hd in params["node_heads"]])
    We = jnp.concatenate([hd["W"] for hd in params["edge_heads"]], axis=1)
    be = jnp.concatenate([hd["b"] for hd in params["edge_heads"]])
    NV = Wn.shape[1]
    WeP = jnp.pad(We, ((0, 0), (0, 16 - We.shape[1])))
    beP = jnp.pad(be, (0, 16 - be.shape[0]))

    layer = params["layers"][-1]
    aggA, aggB = sc_msg_agg(hA, hB, CA, CB, src_p, ci_p, dst_p, zer)
    node_pred_p, Q, Qb = _mlp_last_call(NP, NV)(
        hA, hB, aggA, aggB,
        layer["lin1"]["W"], layer["lin1"]["b"][None, :],
        layer["lin2"]["W"], layer["lin2"]["b"][None, :],
        Wn, bn[None, :], WeP, beP[None, :])

    ep = _edge_pred_kernel(NP, EP)(Q, Qb, src_p, dst_p)

    node_pred = node_pred_p[:N]
    edge_pred = ep[:E, :9]

    return node_pred, edge_pred, node_mask, edge_mask
